# baseline (speedup 1.0000x reference)
import functools

import jax
import jax.numpy as jnp
from jax import lax
from jax.experimental import pallas as pl
from jax.experimental.pallas import tpu as pltpu
from jax.experimental.pallas import tpu_sc as plsc

N = 10000
D = 128
E = 320000

LANES = 128
NW = 32
NB = 8
NG = 80
EPAD = NW * NG * LANES
NSUB = 16
NPAD = 10240
ROWS_PER_SUB = NPAD // NSUB

ROWS_BLK = 1000
GRID_N = N // ROWS_BLK


def _segsum_body(feat, y_hbm, src_hbm, dst_hbm, w_hbm, zero_hbm, out_hbm,
                 src_v, dst_v, w_v, rows_v, *sems):
    acc_sh = sems[-1]
    gsems = sems[0:NB]
    ssems = sems[NB:2 * NB]
    c = lax.axis_index("c")
    s = lax.axis_index("s")
    wid = s * 2 + c

    pltpu.sync_copy(src_hbm.at[wid], src_v)
    pltpu.sync_copy(dst_hbm.at[wid], dst_v)
    pltpu.sync_copy(w_hbm.at[wid], w_v)

    pltpu.sync_copy(zero_hbm.at[pl.ds(s * ROWS_PER_SUB, ROWS_PER_SUB)],
                    acc_sh.at[pl.ds(s * ROWS_PER_SUB, ROWS_PER_SUB)])
    plsc.subcore_barrier()

    def start_gather(j, b):
        pltpu.async_copy(y_hbm.at[src_v.at[j]], rows_v.at[b], gsems[b])

    def wait_gather(j, b):
        pltpu.make_async_copy(y_hbm.at[src_v.at[j]], rows_v.at[b],
                              gsems[b]).wait()

    def start_scatter(j, b):
        pltpu.async_copy(rows_v.at[b], acc_sh.at[dst_v.at[j]], ssems[b],
                         add=True)

    def wait_scatter(j, b):
        pltpu.make_async_copy(rows_v.at[b], acc_sh.at[dst_v.at[j]],
                              ssems[b]).wait()

    def scale_rows(j, b):
        def scale(blk, carry2):
            e0 = blk * 16
            wv = w_v[j, pl.ds(e0, 16)]
            for k in range(16):
                we = wv[k]
                for f0 in range(0, feat, 16):
                    rows_v[b, e0 + k, pl.ds(f0, 16)] = (
                        rows_v[b, e0 + k, pl.ds(f0, 16)] * we)
            return carry2
        lax.fori_loop(0, LANES // 16, scale, 0, unroll=2)

    for b in range(NB):
        start_gather(b, b)

    def round_body(t, carry):
        i = t * NB
        for b in range(NB):
            j = i + b
            wait_gather(j, b)
            scale_rows(j, b)
            start_scatter(j, b)
        for b in range(NB):
            j = i + b
            wait_scatter(j, b)
            start_gather(j + NB, b)
        return carry

    lax.fori_loop(0, NG // NB - 1, round_body, 0)

    for b in range(NB):
        j = NG - NB + b
        wait_gather(j, b)
        scale_rows(j, b)
        start_scatter(j, b)
    for b in range(NB):
        wait_scatter(NG - NB + b, b)
    plsc.subcore_barrier()

    pltpu.sync_copy(acc_sh.at[pl.ds(s * ROWS_PER_SUB, ROWS_PER_SUB)],
                    out_hbm.at[c, pl.ds(s * ROWS_PER_SUB, ROWS_PER_SUB)])


def _make_segsum(feat):
    mesh = plsc.VectorSubcoreMesh(core_axis_name="c", subcore_axis_name="s")
    return pl.kernel(
        functools.partial(_segsum_body, feat),
        out_type=jax.ShapeDtypeStruct((2, NPAD, feat), jnp.float32),
        mesh=mesh,
        compiler_params=pltpu.CompilerParams(use_tc_tiling_on_sc=False),
        scratch_types=[
            pltpu.VMEM((NG, LANES), jnp.int32),
            pltpu.VMEM((NG, LANES), jnp.int32),
            pltpu.VMEM((NG, LANES), jnp.float32),
            pltpu.VMEM((NB, LANES, feat), jnp.float32),
        ] + [pltpu.SemaphoreType.DMA] * (2 * NB) + [
            pltpu.VMEM_SHARED((NPAD, feat), jnp.float32),
        ],
    )


_segsum32 = _make_segsum(32)
_segsum16 = _make_segsum(16)


def _mm_body(x_ref, w_ref, o_ref):
    o_ref[...] = jnp.dot(x_ref[...], w_ref[...],
                         preferred_element_type=jnp.float32)


def _layer1_matmul(x, w1c):
    return pl.pallas_call(
        _mm_body,
        grid=(GRID_N,),
        in_specs=[pl.BlockSpec((ROWS_BLK, D), lambda i: (i, 0)),
                  pl.BlockSpec((D, 64), lambda i: (0, 0))],
        out_specs=pl.BlockSpec((ROWS_BLK, 64), lambda i: (i, 0)),
        out_shape=jax.ShapeDtypeStruct((N, 64), jnp.float32),
    )(x, w1c)


def _mid_body(a0_ref, a1_ref, r1_ref, b1_ref, w_ref, o_ref):
    h = jnp.maximum(a0_ref[...] + a1_ref[...] + r1_ref[...] + b1_ref[...], 0.0)
    o_ref[...] = jnp.dot(h, w_ref[...], preferred_element_type=jnp.float32)


def _mid_layer(a0, a1, r1, b1, w2c):
    return pl.pallas_call(
        _mid_body,
        grid=(GRID_N,),
        in_specs=[pl.BlockSpec((ROWS_BLK, 32), lambda i: (i, 0)),
                  pl.BlockSpec((ROWS_BLK, 32), lambda i: (i, 0)),
                  pl.BlockSpec((ROWS_BLK, 32), lambda i: (i, 0)),
                  pl.BlockSpec((1, 32), lambda i: (0, 0)),
                  pl.BlockSpec((32, 32), lambda i: (0, 0))],
        out_specs=pl.BlockSpec((ROWS_BLK, 32), lambda i: (i, 0)),
        out_shape=jax.ShapeDtypeStruct((N, 32), jnp.float32),
    )(a0, a1, r1, b1, w2c)


def _final_body(a0_ref, a1_ref, oc_ref, x1_ref, b2_ref, wl_ref, bl_ref,
                out_ref, emb_ref):
    t = (a0_ref[...][:, :8] + a1_ref[...][:, :8]
         + oc_ref[...][:, 16:24] + b2_ref[...])
    m = jnp.max(t, axis=1, keepdims=True)
    lse = jnp.log(jnp.sum(jnp.exp(t - m), axis=1, keepdims=True)) + m
    h2 = t - lse
    emb_ref[...] = h2
    s = jnp.sum(h2 * wl_ref[...][:, :8], axis=1, keepdims=True)
    out = s + x1_ref[...] * wl_ref[...][:, 8:9] + bl_ref[...]
    out_ref[...] = jnp.maximum(out, 0.0)


def _final_layer(a0, a1, oc, x1, b2, wl, bl):
    return pl.pallas_call(
        _final_body,
        grid=(GRID_N,),
        in_specs=[pl.BlockSpec((ROWS_BLK, 16), lambda i: (i, 0)),
                  pl.BlockSpec((ROWS_BLK, 16), lambda i: (i, 0)),
                  pl.BlockSpec((ROWS_BLK, 32), lambda i: (i, 0)),
                  pl.BlockSpec((ROWS_BLK, 1), lambda i: (i, 0)),
                  pl.BlockSpec((1, 8), lambda i: (0, 0)),
                  pl.BlockSpec((1, 9), lambda i: (0, 0)),
                  pl.BlockSpec((1, 1), lambda i: (0, 0))],
        out_specs=[pl.BlockSpec((ROWS_BLK, 1), lambda i: (i, 0)),
                   pl.BlockSpec((ROWS_BLK, 8), lambda i: (i, 0))],
        out_shape=[jax.ShapeDtypeStruct((N, 1), jnp.float32),
                   jax.ShapeDtypeStruct((N, 8), jnp.float32)],
    )(a0, a1, oc, x1, b2, wl, bl)


def kernel(x, edge_index, x1, edge_weight, W1_rel, b1_rel, W1_root,
           W2_rel, b2_rel, W2_root, W_lin, b_lin):
    pad = EPAD - E
    srcp = jnp.concatenate(
        [edge_index[0], jnp.zeros((pad,), jnp.int32)]).reshape(NW, NG, LANES)
    dstp = jnp.concatenate(
        [edge_index[1], jnp.zeros((pad,), jnp.int32)]).reshape(NW, NG, LANES)
    wp = jnp.concatenate(
        [edge_weight, jnp.zeros((pad,), jnp.float32)]).reshape(NW, NG, LANES)

    w1c = jnp.concatenate([W1_rel, W1_root], axis=1)
    z1 = _layer1_matmul(x, w1c)
    y1 = z1[:, :32]
    r1 = z1[:, 32:]

    agg1 = _segsum32(y1, srcp, dstp, wp, jnp.zeros((NPAD, 32), jnp.float32))
    agg1 = agg1[:, :N]

    w2c = jnp.concatenate(
        [W2_rel, jnp.zeros((32, 8), jnp.float32),
         W2_root, jnp.zeros((32, 8), jnp.float32)], axis=1)
    oc = _mid_layer(agg1[0], agg1[1], r1, b1_rel.reshape(1, 32), w2c)
    y2p = oc[:, :16]

    agg2 = _segsum16(y2p, srcp, dstp, wp, jnp.zeros((NPAD, 16), jnp.float32))
    agg2 = agg2[:, :N]

    out, emb = _final_layer(agg2[0], agg2[1], oc, x1,
                            b2_rel.reshape(1, 8), W_lin.T,
                            b_lin.reshape(1, 1))
    return (out, emb)


# speedup vs baseline: 1.0062x; 1.0062x over previous
import functools

import jax
import jax.numpy as jnp
from jax import lax
from jax.experimental import pallas as pl
from jax.experimental.pallas import tpu as pltpu
from jax.experimental.pallas import tpu_sc as plsc

N = 10000
D = 128
E = 320000

LANES = 128
NW = 32
NB = 8
NG = 80
EPAD = NW * NG * LANES
NSUB = 16
NPAD = 10240
ROWS_PER_SUB = NPAD // NSUB

ROWS_BLK = 1000
GRID_N = N // ROWS_BLK


def _segsum_body(feat, y_hbm, src_hbm, dst_hbm, w_hbm, zero_hbm, out_hbm,
                 src_v, dst_v, w_v, rows_v, msg_v, *sems):
    acc_sh = sems[-1]
    gsems = sems[0:NB]
    ssems = sems[NB:2 * NB]
    c = lax.axis_index("c")
    s = lax.axis_index("s")
    wid = s * 2 + c

    pltpu.sync_copy(src_hbm.at[wid], src_v)
    pltpu.sync_copy(dst_hbm.at[wid], dst_v)
    pltpu.sync_copy(w_hbm.at[wid], w_v)

    pltpu.sync_copy(zero_hbm.at[pl.ds(s * ROWS_PER_SUB, ROWS_PER_SUB)],
                    acc_sh.at[pl.ds(s * ROWS_PER_SUB, ROWS_PER_SUB)])
    plsc.subcore_barrier()

    def start_gather(j, b):
        pltpu.async_copy(y_hbm.at[src_v.at[j]], rows_v.at[b], gsems[b])

    def wait_gather(j, b):
        pltpu.make_async_copy(y_hbm.at[src_v.at[j]], rows_v.at[b],
                              gsems[b]).wait()

    def start_scatter(j, b):
        pltpu.async_copy(msg_v.at[b], acc_sh.at[dst_v.at[j]], ssems[b],
                         add=True)

    def wait_scatter(j, b):
        pltpu.make_async_copy(msg_v.at[b], acc_sh.at[dst_v.at[j]],
                              ssems[b]).wait()

    def scale_rows(j, b):
        def scale(blk, carry2):
            e0 = blk * 16
            wv = w_v[j, pl.ds(e0, 16)]
            for k in range(16):
                we = wv[k]
                for f0 in range(0, feat, 16):
                    msg_v[b, e0 + k, pl.ds(f0, 16)] = (
                        rows_v[b, e0 + k, pl.ds(f0, 16)] * we)
            return carry2
        lax.fori_loop(0, LANES // 16, scale, 0, unroll=4)

    for b in range(NB):
        start_gather(b, b)

    def round_body(t, carry):
        i = t * NB
        for b in range(NB):
            j = i + b
            wait_gather(j, b)
            scale_rows(j, b)
            start_scatter(j, b)
        for b in range(NB):
            j = i + b
            wait_scatter(j, b)
            start_gather(j + NB, b)
        return carry

    lax.fori_loop(0, NG // NB - 1, round_body, 0)

    for b in range(NB):
        j = NG - NB + b
        wait_gather(j, b)
        scale_rows(j, b)
        start_scatter(j, b)
    for b in range(NB):
        wait_scatter(NG - NB + b, b)
    plsc.subcore_barrier()

    pltpu.sync_copy(acc_sh.at[pl.ds(s * ROWS_PER_SUB, ROWS_PER_SUB)],
                    out_hbm.at[c, pl.ds(s * ROWS_PER_SUB, ROWS_PER_SUB)])


def _make_segsum(feat):
    mesh = plsc.VectorSubcoreMesh(core_axis_name="c", subcore_axis_name="s")
    return pl.kernel(
        functools.partial(_segsum_body, feat),
        out_type=jax.ShapeDtypeStruct((2, NPAD, feat), jnp.float32),
        mesh=mesh,
        compiler_params=pltpu.CompilerParams(use_tc_tiling_on_sc=False),
        scratch_types=[
            pltpu.VMEM((NG, LANES), jnp.int32),
            pltpu.VMEM((NG, LANES), jnp.int32),
            pltpu.VMEM((NG, LANES), jnp.float32),
            pltpu.VMEM((NB, LANES, feat), jnp.float32),
            pltpu.VMEM((NB, LANES, feat), jnp.float32),
        ] + [pltpu.SemaphoreType.DMA] * (2 * NB) + [
            pltpu.VMEM_SHARED((NPAD, feat), jnp.float32),
        ],
    )


_segsum32 = _make_segsum(32)
_segsum16 = _make_segsum(16)


def _mm_body(x_ref, w_ref, o_ref):
    o_ref[...] = jnp.dot(x_ref[...], w_ref[...],
                         preferred_element_type=jnp.float32)


def _layer1_matmul(x, w1c):
    return pl.pallas_call(
        _mm_body,
        grid=(GRID_N,),
        in_specs=[pl.BlockSpec((ROWS_BLK, D), lambda i: (i, 0)),
                  pl.BlockSpec((D, 64), lambda i: (0, 0))],
        out_specs=pl.BlockSpec((ROWS_BLK, 64), lambda i: (i, 0)),
        out_shape=jax.ShapeDtypeStruct((N, 64), jnp.float32),
    )(x, w1c)


def _mid_body(a0_ref, a1_ref, r1_ref, b1_ref, w_ref, o_ref):
    h = jnp.maximum(a0_ref[...] + a1_ref[...] + r1_ref[...] + b1_ref[...], 0.0)
    o_ref[...] = jnp.dot(h, w_ref[...], preferred_element_type=jnp.float32)


def _mid_layer(a0, a1, r1, b1, w2c):
    return pl.pallas_call(
        _mid_body,
        grid=(GRID_N,),
        in_specs=[pl.BlockSpec((ROWS_BLK, 32), lambda i: (i, 0)),
                  pl.BlockSpec((ROWS_BLK, 32), lambda i: (i, 0)),
                  pl.BlockSpec((ROWS_BLK, 32), lambda i: (i, 0)),
                  pl.BlockSpec((1, 32), lambda i: (0, 0)),
                  pl.BlockSpec((32, 32), lambda i: (0, 0))],
        out_specs=pl.BlockSpec((ROWS_BLK, 32), lambda i: (i, 0)),
        out_shape=jax.ShapeDtypeStruct((N, 32), jnp.float32),
    )(a0, a1, r1, b1, w2c)


def _final_body(a0_ref, a1_ref, oc_ref, x1_ref, b2_ref, wl_ref, bl_ref,
                out_ref, emb_ref):
    t = (a0_ref[...][:, :8] + a1_ref[...][:, :8]
         + oc_ref[...][:, 16:24] + b2_ref[...])
    m = jnp.max(t, axis=1, keepdims=True)
    lse = jnp.log(jnp.sum(jnp.exp(t - m), axis=1, keepdims=True)) + m
    h2 = t - lse
    emb_ref[...] = h2
    s = jnp.sum(h2 * wl_ref[...][:, :8], axis=1, keepdims=True)
    out = s + x1_ref[...] * wl_ref[...][:, 8:9] + bl_ref[...]
    out_ref[...] = jnp.maximum(out, 0.0)


def _final_layer(a0, a1, oc, x1, b2, wl, bl):
    return pl.pallas_call(
        _final_body,
        grid=(GRID_N,),
        in_specs=[pl.BlockSpec((ROWS_BLK, 16), lambda i: (i, 0)),
                  pl.BlockSpec((ROWS_BLK, 16), lambda i: (i, 0)),
                  pl.BlockSpec((ROWS_BLK, 32), lambda i: (i, 0)),
                  pl.BlockSpec((ROWS_BLK, 1), lambda i: (i, 0)),
                  pl.BlockSpec((1, 8), lambda i: (0, 0)),
                  pl.BlockSpec((1, 9), lambda i: (0, 0)),
                  pl.BlockSpec((1, 1), lambda i: (0, 0))],
        out_specs=[pl.BlockSpec((ROWS_BLK, 1), lambda i: (i, 0)),
                   pl.BlockSpec((ROWS_BLK, 8), lambda i: (i, 0))],
        out_shape=[jax.ShapeDtypeStruct((N, 1), jnp.float32),
                   jax.ShapeDtypeStruct((N, 8), jnp.float32)],
    )(a0, a1, oc, x1, b2, wl, bl)


def kernel(x, edge_index, x1, edge_weight, W1_rel, b1_rel, W1_root,
           W2_rel, b2_rel, W2_root, W_lin, b_lin):
    pad = EPAD - E
    srcp = jnp.concatenate(
        [edge_index[0], jnp.zeros((pad,), jnp.int32)]).reshape(NW, NG, LANES)
    dstp = jnp.concatenate(
        [edge_index[1], jnp.zeros((pad,), jnp.int32)]).reshape(NW, NG, LANES)
    wp = jnp.concatenate(
        [edge_weight, jnp.zeros((pad,), jnp.float32)]).reshape(NW, NG, LANES)

    w1c = jnp.concatenate([W1_rel, W1_root], axis=1)
    z1 = _layer1_matmul(x, w1c)
    y1 = z1[:, :32]
    r1 = z1[:, 32:]

    agg1 = _segsum32(y1, srcp, dstp, wp, jnp.zeros((NPAD, 32), jnp.float32))
    agg1 = agg1[:, :N]

    w2c = jnp.concatenate(
        [W2_rel, jnp.zeros((32, 8), jnp.float32),
         W2_root, jnp.zeros((32, 8), jnp.float32)], axis=1)
    oc = _mid_layer(agg1[0], agg1[1], r1, b1_rel.reshape(1, 32), w2c)
    y2p = oc[:, :16]

    agg2 = _segsum16(y2p, srcp, dstp, wp, jnp.zeros((NPAD, 16), jnp.float32))
    agg2 = agg2[:, :N]

    out, emb = _final_layer(agg2[0], agg2[1], oc, x1,
                            b2_rel.reshape(1, 8), W_lin.T,
                            b_lin.reshape(1, 1))
    return (out, emb)


# speedup vs baseline: 1.5149x; 1.5056x over previous
import functools

import jax
import jax.numpy as jnp
from jax import lax
from jax.experimental import pallas as pl
from jax.experimental.pallas import tpu as pltpu
from jax.experimental.pallas import tpu_sc as plsc

N = 10000
D = 128
E = 320000

LANES = 128
NW = 32
NB = 4
NG = 80
EPAD = NW * NG * LANES
NSUB = 16
NPAD = 10240
ROWS_PER_SUB = NPAD // NSUB

ROWS_BLK = 1000
GRID_N = N // ROWS_BLK


def _segsum_body(feat, y_hbm, src_hbm, dst_hbm, w_hbm, zero_hbm, out_hbm,
                 src_v, dst_v, w_v, rows_v, msg_v, *sems):
    acc_sh = sems[-2]
    y_sh = sems[-1]
    gsems = sems[0:NB]
    ssems = sems[NB:2 * NB]
    c = lax.axis_index("c")
    s = lax.axis_index("s")
    wid = s * 2 + c

    pltpu.sync_copy(src_hbm.at[wid], src_v)
    pltpu.sync_copy(dst_hbm.at[wid], dst_v)
    pltpu.sync_copy(w_hbm.at[wid], w_v)

    pltpu.sync_copy(zero_hbm.at[pl.ds(s * ROWS_PER_SUB, ROWS_PER_SUB)],
                    acc_sh.at[pl.ds(s * ROWS_PER_SUB, ROWS_PER_SUB)])
    pltpu.sync_copy(y_hbm.at[pl.ds(s * (N // NSUB), N // NSUB)],
                    y_sh.at[pl.ds(s * (N // NSUB), N // NSUB)])
    plsc.subcore_barrier()

    def start_gather(j, b):
        pltpu.async_copy(y_sh.at[src_v.at[j]], rows_v.at[b], gsems[b])

    def wait_gather(j, b):
        pltpu.make_async_copy(y_sh.at[src_v.at[j]], rows_v.at[b],
                              gsems[b]).wait()

    def start_scatter(j, b):
        pltpu.async_copy(msg_v.at[b], acc_sh.at[dst_v.at[j]], ssems[b],
                         add=True)

    def wait_scatter(j, b):
        pltpu.make_async_copy(msg_v.at[b], acc_sh.at[dst_v.at[j]],
                              ssems[b]).wait()

    def scale_rows(j, b):
        def scale(blk, carry2):
            e0 = blk * 16
            wv = w_v[j, pl.ds(e0, 16)]
            for k in range(16):
                we = wv[k]
                for f0 in range(0, feat, 16):
                    msg_v[b, e0 + k, pl.ds(f0, 16)] = (
                        rows_v[b, e0 + k, pl.ds(f0, 16)] * we)
            return carry2
        lax.fori_loop(0, LANES // 16, scale, 0, unroll=4)

    def round_body(j, carry):
        start_gather(j, 0)
        wait_gather(j, 0)
        scale_rows(j, 0)
        start_scatter(j, 0)
        wait_scatter(j, 0)
        return carry

    lax.fori_loop(0, NG, round_body, 0)
    plsc.subcore_barrier()

    pltpu.sync_copy(acc_sh.at[pl.ds(s * ROWS_PER_SUB, ROWS_PER_SUB)],
                    out_hbm.at[c, pl.ds(s * ROWS_PER_SUB, ROWS_PER_SUB)])


def _make_segsum(feat):
    mesh = plsc.VectorSubcoreMesh(core_axis_name="c", subcore_axis_name="s")
    return pl.kernel(
        functools.partial(_segsum_body, feat),
        out_type=jax.ShapeDtypeStruct((2, NPAD, feat), jnp.float32),
        mesh=mesh,
        compiler_params=pltpu.CompilerParams(use_tc_tiling_on_sc=False),
        scratch_types=[
            pltpu.VMEM((NG, LANES), jnp.int32),
            pltpu.VMEM((NG, LANES), jnp.int32),
            pltpu.VMEM((NG, LANES), jnp.float32),
            pltpu.VMEM((NB, LANES, feat), jnp.float32),
            pltpu.VMEM((NB, LANES, feat), jnp.float32),
        ] + [pltpu.SemaphoreType.DMA] * (2 * NB) + [
            pltpu.VMEM_SHARED((NPAD, feat), jnp.float32),
            pltpu.VMEM_SHARED((N, feat), jnp.float32),
        ],
    )


_segsum32 = _make_segsum(32)
_segsum16 = _make_segsum(16)


def _mm_body(x_ref, w_ref, o_ref):
    o_ref[...] = jnp.dot(x_ref[...], w_ref[...],
                         preferred_element_type=jnp.float32)


def _layer1_matmul(x, w1c):
    return pl.pallas_call(
        _mm_body,
        grid=(GRID_N,),
        in_specs=[pl.BlockSpec((ROWS_BLK, D), lambda i: (i, 0)),
                  pl.BlockSpec((D, 64), lambda i: (0, 0))],
        out_specs=pl.BlockSpec((ROWS_BLK, 64), lambda i: (i, 0)),
        out_shape=jax.ShapeDtypeStruct((N, 64), jnp.float32),
    )(x, w1c)


def _mid_body(a0_ref, a1_ref, r1_ref, b1_ref, w_ref, o_ref):
    h = jnp.maximum(a0_ref[...] + a1_ref[...] + r1_ref[...] + b1_ref[...], 0.0)
    o_ref[...] = jnp.dot(h, w_ref[...], preferred_element_type=jnp.float32)


def _mid_layer(a0, a1, r1, b1, w2c):
    return pl.pallas_call(
        _mid_body,
        grid=(GRID_N,),
        in_specs=[pl.BlockSpec((ROWS_BLK, 32), lambda i: (i, 0)),
                  pl.BlockSpec((ROWS_BLK, 32), lambda i: (i, 0)),
                  pl.BlockSpec((ROWS_BLK, 32), lambda i: (i, 0)),
                  pl.BlockSpec((1, 32), lambda i: (0, 0)),
                  pl.BlockSpec((32, 32), lambda i: (0, 0))],
        out_specs=pl.BlockSpec((ROWS_BLK, 32), lambda i: (i, 0)),
        out_shape=jax.ShapeDtypeStruct((N, 32), jnp.float32),
    )(a0, a1, r1, b1, w2c)


def _final_body(a0_ref, a1_ref, oc_ref, x1_ref, b2_ref, wl_ref, bl_ref,
                out_ref, emb_ref):
    t = (a0_ref[...][:, :8] + a1_ref[...][:, :8]
         + oc_ref[...][:, 16:24] + b2_ref[...])
    m = jnp.max(t, axis=1, keepdims=True)
    lse = jnp.log(jnp.sum(jnp.exp(t - m), axis=1, keepdims=True)) + m
    h2 = t - lse
    emb_ref[...] = h2
    s = jnp.sum(h2 * wl_ref[...][:, :8], axis=1, keepdims=True)
    out = s + x1_ref[...] * wl_ref[...][:, 8:9] + bl_ref[...]
    out_ref[...] = jnp.maximum(out, 0.0)


def _final_layer(a0, a1, oc, x1, b2, wl, bl):
    return pl.pallas_call(
        _final_body,
        grid=(GRID_N,),
        in_specs=[pl.BlockSpec((ROWS_BLK, 16), lambda i: (i, 0)),
                  pl.BlockSpec((ROWS_BLK, 16), lambda i: (i, 0)),
                  pl.BlockSpec((ROWS_BLK, 32), lambda i: (i, 0)),
                  pl.BlockSpec((ROWS_BLK, 1), lambda i: (i, 0)),
                  pl.BlockSpec((1, 8), lambda i: (0, 0)),
                  pl.BlockSpec((1, 9), lambda i: (0, 0)),
                  pl.BlockSpec((1, 1), lambda i: (0, 0))],
        out_specs=[pl.BlockSpec((ROWS_BLK, 1), lambda i: (i, 0)),
                   pl.BlockSpec((ROWS_BLK, 8), lambda i: (i, 0))],
        out_shape=[jax.ShapeDtypeStruct((N, 1), jnp.float32),
                   jax.ShapeDtypeStruct((N, 8), jnp.float32)],
    )(a0, a1, oc, x1, b2, wl, bl)


def kernel(x, edge_index, x1, edge_weight, W1_rel, b1_rel, W1_root,
           W2_rel, b2_rel, W2_root, W_lin, b_lin):
    pad = EPAD - E
    srcp = jnp.concatenate(
        [edge_index[0], jnp.zeros((pad,), jnp.int32)]).reshape(NW, NG, LANES)
    dstp = jnp.concatenate(
        [edge_index[1], jnp.zeros((pad,), jnp.int32)]).reshape(NW, NG, LANES)
    wp = jnp.concatenate(
        [edge_weight, jnp.zeros((pad,), jnp.float32)]).reshape(NW, NG, LANES)

    w1c = jnp.concatenate([W1_rel, W1_root], axis=1)
    z1 = _layer1_matmul(x, w1c)
    y1 = z1[:, :32]
    r1 = z1[:, 32:]

    agg1 = _segsum32(y1, srcp, dstp, wp, jnp.zeros((NPAD, 32), jnp.float32))
    agg1 = agg1[:, :N]

    w2c = jnp.concatenate(
        [W2_rel, jnp.zeros((32, 8), jnp.float32),
         W2_root, jnp.zeros((32, 8), jnp.float32)], axis=1)
    oc = _mid_layer(agg1[0], agg1[1], r1, b1_rel.reshape(1, 32), w2c)
    y2p = oc[:, :16]

    agg2 = _segsum16(y2p, srcp, dstp, wp, jnp.zeros((NPAD, 16), jnp.float32))
    agg2 = agg2[:, :N]

    out, emb = _final_layer(agg2[0], agg2[1], oc, x1,
                            b2_rel.reshape(1, 8), W_lin.T,
                            b_lin.reshape(1, 1))
    return (out, emb)


# speedup vs baseline: 1.5186x; 1.0025x over previous
import functools

import jax
import jax.numpy as jnp
from jax import lax
from jax.experimental import pallas as pl
from jax.experimental.pallas import tpu as pltpu
from jax.experimental.pallas import tpu_sc as plsc

N = 10000
D = 128
E = 320000

LANES = 128
NW = 32
NB = 4
NG = 80
EPAD = NW * NG * LANES
NSUB = 16
NPAD = 10240
ROWS_PER_SUB = NPAD // NSUB

ROWS_BLK = 1000
GRID_N = N // ROWS_BLK


def _segsum_body(feat, y_hbm, src_hbm, dst_hbm, w_hbm, zero_hbm, out_hbm,
                 src_v, dst_v, w_v, rows_v, msg_v, gsem, ssem, acc_sh, y_sh):
    c = lax.axis_index("c")
    s = lax.axis_index("s")
    wid = s * 2 + c

    pltpu.sync_copy(src_hbm.at[wid], src_v)
    pltpu.sync_copy(dst_hbm.at[wid], dst_v)
    pltpu.sync_copy(w_hbm.at[wid], w_v)

    pltpu.sync_copy(zero_hbm.at[pl.ds(s * ROWS_PER_SUB, ROWS_PER_SUB)],
                    acc_sh.at[pl.ds(s * ROWS_PER_SUB, ROWS_PER_SUB)])
    pltpu.sync_copy(y_hbm.at[pl.ds(s * (N // NSUB), N // NSUB)],
                    y_sh.at[pl.ds(s * (N // NSUB), N // NSUB)])
    plsc.subcore_barrier()

    def start_gather(j):
        pltpu.async_copy(y_sh.at[src_v.at[j]], rows_v, gsem)

    def wait_gather(j):
        pltpu.make_async_copy(y_sh.at[src_v.at[j]], rows_v, gsem).wait()

    def start_scatter(j):
        pltpu.async_copy(msg_v, acc_sh.at[dst_v.at[j]], ssem, add=True)

    def wait_scatter(j):
        pltpu.make_async_copy(msg_v, acc_sh.at[dst_v.at[j]], ssem).wait()

    def scale_rows(j):
        def scale(blk, carry2):
            e0 = blk * 16
            wv = w_v[j, pl.ds(e0, 16)]
            for k in range(16):
                we = wv[k]
                for f0 in range(0, feat, 16):
                    msg_v[e0 + k, pl.ds(f0, 16)] = (
                        rows_v[e0 + k, pl.ds(f0, 16)] * we)
            return carry2
        lax.fori_loop(0, LANES // 16, scale, 0, unroll=4)

    def body(j, carry):
        start_gather(j)
        wait_gather(j)
        scale_rows(j)
        start_scatter(j)
        wait_scatter(j)
        return carry

    lax.fori_loop(0, NG, body, 0)
    plsc.subcore_barrier()

    pltpu.sync_copy(acc_sh.at[pl.ds(s * ROWS_PER_SUB, ROWS_PER_SUB)],
                    out_hbm.at[c, pl.ds(s * ROWS_PER_SUB, ROWS_PER_SUB)])


def _make_segsum(feat):
    mesh = plsc.VectorSubcoreMesh(core_axis_name="c", subcore_axis_name="s")
    return pl.kernel(
        functools.partial(_segsum_body, feat),
        out_type=jax.ShapeDtypeStruct((2, NPAD, feat), jnp.float32),
        mesh=mesh,
        compiler_params=pltpu.CompilerParams(use_tc_tiling_on_sc=False),
        scratch_types=[
            pltpu.VMEM((NG + 1, LANES), jnp.int32),
            pltpu.VMEM((NG, LANES), jnp.int32),
            pltpu.VMEM((NG, LANES), jnp.float32),
            pltpu.VMEM((LANES, feat), jnp.float32),
            pltpu.VMEM((LANES, feat), jnp.float32),
            pltpu.SemaphoreType.DMA,
            pltpu.SemaphoreType.DMA,
            pltpu.VMEM_SHARED((NPAD, feat), jnp.float32),
            pltpu.VMEM_SHARED((N, feat), jnp.float32),
        ],
    )


_segsum32 = _make_segsum(32)
_segsum16 = _make_segsum(16)


def _mm_body(x_ref, w_ref, o_ref):
    o_ref[...] = jnp.dot(x_ref[...], w_ref[...],
                         preferred_element_type=jnp.float32)


def _layer1_matmul(x, w1c):
    return pl.pallas_call(
        _mm_body,
        grid=(GRID_N,),
        in_specs=[pl.BlockSpec((ROWS_BLK, D), lambda i: (i, 0)),
                  pl.BlockSpec((D, 64), lambda i: (0, 0))],
        out_specs=pl.BlockSpec((ROWS_BLK, 64), lambda i: (i, 0)),
        out_shape=jax.ShapeDtypeStruct((N, 64), jnp.float32),
    )(x, w1c)


def _mid_body(a0_ref, a1_ref, r1_ref, b1_ref, w_ref, o_ref):
    h = jnp.maximum(a0_ref[...] + a1_ref[...] + r1_ref[...] + b1_ref[...], 0.0)
    o_ref[...] = jnp.dot(h, w_ref[...], preferred_element_type=jnp.float32)


def _mid_layer(a0, a1, r1, b1, w2c):
    return pl.pallas_call(
        _mid_body,
        grid=(GRID_N,),
        in_specs=[pl.BlockSpec((ROWS_BLK, 32), lambda i: (i, 0)),
                  pl.BlockSpec((ROWS_BLK, 32), lambda i: (i, 0)),
                  pl.BlockSpec((ROWS_BLK, 32), lambda i: (i, 0)),
                  pl.BlockSpec((1, 32), lambda i: (0, 0)),
                  pl.BlockSpec((32, 32), lambda i: (0, 0))],
        out_specs=pl.BlockSpec((ROWS_BLK, 32), lambda i: (i, 0)),
        out_shape=jax.ShapeDtypeStruct((N, 32), jnp.float32),
    )(a0, a1, r1, b1, w2c)


def _final_body(a0_ref, a1_ref, oc_ref, x1_ref, b2_ref, wl_ref, bl_ref,
                out_ref, emb_ref):
    t = (a0_ref[...][:, :8] + a1_ref[...][:, :8]
         + oc_ref[...][:, 16:24] + b2_ref[...])
    m = jnp.max(t, axis=1, keepdims=True)
    lse = jnp.log(jnp.sum(jnp.exp(t - m), axis=1, keepdims=True)) + m
    h2 = t - lse
    emb_ref[...] = h2
    s = jnp.sum(h2 * wl_ref[...][:, :8], axis=1, keepdims=True)
    out = s + x1_ref[...] * wl_ref[...][:, 8:9] + bl_ref[...]
    out_ref[...] = jnp.maximum(out, 0.0)


def _final_layer(a0, a1, oc, x1, b2, wl, bl):
    return pl.pallas_call(
        _final_body,
        grid=(GRID_N,),
        in_specs=[pl.BlockSpec((ROWS_BLK, 16), lambda i: (i, 0)),
                  pl.BlockSpec((ROWS_BLK, 16), lambda i: (i, 0)),
                  pl.BlockSpec((ROWS_BLK, 32), lambda i: (i, 0)),
                  pl.BlockSpec((ROWS_BLK, 1), lambda i: (i, 0)),
                  pl.BlockSpec((1, 8), lambda i: (0, 0)),
                  pl.BlockSpec((1, 9), lambda i: (0, 0)),
                  pl.BlockSpec((1, 1), lambda i: (0, 0))],
        out_specs=[pl.BlockSpec((ROWS_BLK, 1), lambda i: (i, 0)),
                   pl.BlockSpec((ROWS_BLK, 8), lambda i: (i, 0))],
        out_shape=[jax.ShapeDtypeStruct((N, 1), jnp.float32),
                   jax.ShapeDtypeStruct((N, 8), jnp.float32)],
    )(a0, a1, oc, x1, b2, wl, bl)


def kernel(x, edge_index, x1, edge_weight, W1_rel, b1_rel, W1_root,
           W2_rel, b2_rel, W2_root, W_lin, b_lin):
    pad = EPAD - E
    srcp = jnp.concatenate(
        [edge_index[0], jnp.zeros((pad,), jnp.int32)]).reshape(NW, NG, LANES)
    srcp = jnp.concatenate(
        [srcp, jnp.zeros((NW, 1, LANES), jnp.int32)], axis=1)
    dstp = jnp.concatenate(
        [edge_index[1], jnp.zeros((pad,), jnp.int32)]).reshape(NW, NG, LANES)
    wp = jnp.concatenate(
        [edge_weight, jnp.zeros((pad,), jnp.float32)]).reshape(NW, NG, LANES)

    w1c = jnp.concatenate([W1_rel, W1_root], axis=1)
    z1 = _layer1_matmul(x, w1c)
    y1 = z1[:, :32]
    r1 = z1[:, 32:]

    agg1 = _segsum32(y1, srcp, dstp, wp, jnp.zeros((NPAD, 32), jnp.float32))
    agg1 = agg1[:, :N]

    w2c = jnp.concatenate(
        [W2_rel, jnp.zeros((32, 8), jnp.float32),
         W2_root, jnp.zeros((32, 8), jnp.float32)], axis=1)
    oc = _mid_layer(agg1[0], agg1[1], r1, b1_rel.reshape(1, 32), w2c)
    y2p = oc[:, :16]

    agg2 = _segsum16(y2p, srcp, dstp, wp, jnp.zeros((NPAD, 16), jnp.float32))
    agg2 = agg2[:, :N]

    out, emb = _final_layer(agg2[0], agg2[1], oc, x1,
                            b2_rel.reshape(1, 8), W_lin.T,
                            b_lin.reshape(1, 1))
    return (out, emb)


# speedup vs baseline: 1.5945x; 1.0499x over previous
import functools

import jax
import jax.numpy as jnp
from jax import lax
from jax.experimental import pallas as pl
from jax.experimental.pallas import tpu as pltpu
from jax.experimental.pallas import tpu_sc as plsc

N = 10000
D = 128
E = 320000

LANES = 128
NW = 32
NB = 4
NG = 80
EPAD = NW * NG * LANES
NSUB = 16
NPAD = 10240
ROWS_PER_SUB = NPAD // NSUB

ROWS_BLK = 1000
GRID_N = N // ROWS_BLK


def _segsum_body(feat, y_hbm, src_hbm, dst_hbm, w_hbm, zero_hbm, out_hbm,
                 src_v, dst_v, w_v, rows_v, msg_v, *rest):
    K = 4
    gsems = rest[0:K]
    ssems = rest[K:2 * K]
    acc_sh = rest[2 * K]
    y_sh = rest[2 * K + 1]
    c = lax.axis_index("c")
    s = lax.axis_index("s")
    wid = s * 2 + c

    pltpu.sync_copy(src_hbm.at[wid], src_v)
    pltpu.sync_copy(dst_hbm.at[wid], dst_v)
    pltpu.sync_copy(w_hbm.at[wid], w_v)

    pltpu.sync_copy(zero_hbm.at[pl.ds(s * ROWS_PER_SUB, ROWS_PER_SUB)],
                    acc_sh.at[pl.ds(s * ROWS_PER_SUB, ROWS_PER_SUB)])
    pltpu.sync_copy(y_hbm.at[pl.ds(s * (N // NSUB), N // NSUB)],
                    y_sh.at[pl.ds(s * (N // NSUB), N // NSUB)])
    plsc.subcore_barrier()

    def start_gather(j, b):
        pltpu.async_copy(y_sh.at[src_v.at[j]], rows_v.at[b], gsems[b])

    def wait_gather(j, b):
        pltpu.make_async_copy(y_sh.at[src_v.at[j]], rows_v.at[b], gsems[b]).wait()

    def start_scatter(j, b):
        pltpu.async_copy(msg_v.at[b], acc_sh.at[dst_v.at[j]], ssems[b], add=True)

    def wait_scatter(j, b):
        pltpu.make_async_copy(msg_v.at[b], acc_sh.at[dst_v.at[j]], ssems[b]).wait()

    def scale_rows(j, b):
        def scale(blk, carry2):
            e0 = blk * 16
            wv = w_v[j, pl.ds(e0, 16)]
            for k in range(16):
                we = wv[k]
                for f0 in range(0, feat, 16):
                    msg_v[b, e0 + k, pl.ds(f0, 16)] = (
                        rows_v[b, e0 + k, pl.ds(f0, 16)] * we)
            return carry2
        lax.fori_loop(0, LANES // 16, scale, 0, unroll=4)

    def body(q, carry):
        j0 = q * K
        for b in range(K):
            start_gather(j0 + b, b)
        for b in range(K):
            wait_gather(j0 + b, b)
            scale_rows(j0 + b, b)
        for b in range(K):
            start_scatter(j0 + b, b)
        for b in range(K):
            wait_scatter(j0 + b, b)
        return carry

    lax.fori_loop(0, NG // K, body, 0)
    plsc.subcore_barrier()

    pltpu.sync_copy(acc_sh.at[pl.ds(s * ROWS_PER_SUB, ROWS_PER_SUB)],
                    out_hbm.at[c, pl.ds(s * ROWS_PER_SUB, ROWS_PER_SUB)])


def _make_segsum(feat):
    mesh = plsc.VectorSubcoreMesh(core_axis_name="c", subcore_axis_name="s")
    return pl.kernel(
        functools.partial(_segsum_body, feat),
        out_type=jax.ShapeDtypeStruct((2, NPAD, feat), jnp.float32),
        mesh=mesh,
        compiler_params=pltpu.CompilerParams(use_tc_tiling_on_sc=False),
        scratch_types=[
            pltpu.VMEM((NG + 1, LANES), jnp.int32),
            pltpu.VMEM((NG, LANES), jnp.int32),
            pltpu.VMEM((NG, LANES), jnp.float32),
            pltpu.VMEM((4, LANES, feat), jnp.float32),
            pltpu.VMEM((4, LANES, feat), jnp.float32),
        ] + [pltpu.SemaphoreType.DMA] * 8 + [
            pltpu.VMEM_SHARED((NPAD, feat), jnp.float32),
            pltpu.VMEM_SHARED((N, feat), jnp.float32),
        ],
    )


_segsum32 = _make_segsum(32)
_segsum16 = _make_segsum(16)


def _mm_body(x_ref, w_ref, o_ref):
    o_ref[...] = jnp.dot(x_ref[...], w_ref[...],
                         preferred_element_type=jnp.float32)


def _layer1_matmul(x, w1c):
    return pl.pallas_call(
        _mm_body,
        grid=(GRID_N,),
        in_specs=[pl.BlockSpec((ROWS_BLK, D), lambda i: (i, 0)),
                  pl.BlockSpec((D, 64), lambda i: (0, 0))],
        out_specs=pl.BlockSpec((ROWS_BLK, 64), lambda i: (i, 0)),
        out_shape=jax.ShapeDtypeStruct((N, 64), jnp.float32),
    )(x, w1c)


def _mid_body(a0_ref, a1_ref, r1_ref, b1_ref, w_ref, o_ref):
    h = jnp.maximum(a0_ref[...] + a1_ref[...] + r1_ref[...] + b1_ref[...], 0.0)
    o_ref[...] = jnp.dot(h, w_ref[...], preferred_element_type=jnp.float32)


def _mid_layer(a0, a1, r1, b1, w2c):
    return pl.pallas_call(
        _mid_body,
        grid=(GRID_N,),
        in_specs=[pl.BlockSpec((ROWS_BLK, 32), lambda i: (i, 0)),
                  pl.BlockSpec((ROWS_BLK, 32), lambda i: (i, 0)),
                  pl.BlockSpec((ROWS_BLK, 32), lambda i: (i, 0)),
                  pl.BlockSpec((1, 32), lambda i: (0, 0)),
                  pl.BlockSpec((32, 32), lambda i: (0, 0))],
        out_specs=pl.BlockSpec((ROWS_BLK, 32), lambda i: (i, 0)),
        out_shape=jax.ShapeDtypeStruct((N, 32), jnp.float32),
    )(a0, a1, r1, b1, w2c)


def _final_body(a0_ref, a1_ref, oc_ref, x1_ref, b2_ref, wl_ref, bl_ref,
                out_ref, emb_ref):
    t = (a0_ref[...][:, :8] + a1_ref[...][:, :8]
         + oc_ref[...][:, 16:24] + b2_ref[...])
    m = jnp.max(t, axis=1, keepdims=True)
    lse = jnp.log(jnp.sum(jnp.exp(t - m), axis=1, keepdims=True)) + m
    h2 = t - lse
    emb_ref[...] = h2
    s = jnp.sum(h2 * wl_ref[...][:, :8], axis=1, keepdims=True)
    out = s + x1_ref[...] * wl_ref[...][:, 8:9] + bl_ref[...]
    out_ref[...] = jnp.maximum(out, 0.0)


def _final_layer(a0, a1, oc, x1, b2, wl, bl):
    return pl.pallas_call(
        _final_body,
        grid=(GRID_N,),
        in_specs=[pl.BlockSpec((ROWS_BLK, 16), lambda i: (i, 0)),
                  pl.BlockSpec((ROWS_BLK, 16), lambda i: (i, 0)),
                  pl.BlockSpec((ROWS_BLK, 32), lambda i: (i, 0)),
                  pl.BlockSpec((ROWS_BLK, 1), lambda i: (i, 0)),
                  pl.BlockSpec((1, 8), lambda i: (0, 0)),
                  pl.BlockSpec((1, 9), lambda i: (0, 0)),
                  pl.BlockSpec((1, 1), lambda i: (0, 0))],
        out_specs=[pl.BlockSpec((ROWS_BLK, 1), lambda i: (i, 0)),
                   pl.BlockSpec((ROWS_BLK, 8), lambda i: (i, 0))],
        out_shape=[jax.ShapeDtypeStruct((N, 1), jnp.float32),
                   jax.ShapeDtypeStruct((N, 8), jnp.float32)],
    )(a0, a1, oc, x1, b2, wl, bl)


def kernel(x, edge_index, x1, edge_weight, W1_rel, b1_rel, W1_root,
           W2_rel, b2_rel, W2_root, W_lin, b_lin):
    pad = EPAD - E
    srcp = jnp.concatenate(
        [edge_index[0], jnp.zeros((pad,), jnp.int32)]).reshape(NW, NG, LANES)
    srcp = jnp.concatenate(
        [srcp, jnp.zeros((NW, 1, LANES), jnp.int32)], axis=1)
    dstp = jnp.concatenate(
        [edge_index[1], jnp.zeros((pad,), jnp.int32)]).reshape(NW, NG, LANES)
    wp = jnp.concatenate(
        [edge_weight, jnp.zeros((pad,), jnp.float32)]).reshape(NW, NG, LANES)

    w1c = jnp.concatenate([W1_rel, W1_root], axis=1)
    z1 = _layer1_matmul(x, w1c)
    y1 = z1[:, :32]
    r1 = z1[:, 32:]

    agg1 = _segsum32(y1, srcp, dstp, wp, jnp.zeros((NPAD, 32), jnp.float32))
    agg1 = agg1[:, :N]

    w2c = jnp.concatenate(
        [W2_rel, jnp.zeros((32, 8), jnp.float32),
         W2_root, jnp.zeros((32, 8), jnp.float32)], axis=1)
    oc = _mid_layer(agg1[0], agg1[1], r1, b1_rel.reshape(1, 32), w2c)
    y2p = oc[:, :16]

    agg2 = _segsum16(y2p, srcp, dstp, wp, jnp.zeros((NPAD, 16), jnp.float32))
    agg2 = agg2[:, :N]

    out, emb = _final_layer(agg2[0], agg2[1], oc, x1,
                            b2_rel.reshape(1, 8), W_lin.T,
                            b_lin.reshape(1, 1))
    return (out, emb)


# speedup vs baseline: 1.6238x; 1.0184x over previous
import functools

import jax
import jax.numpy as jnp
from jax import lax
from jax.experimental import pallas as pl
from jax.experimental.pallas import tpu as pltpu
from jax.experimental.pallas import tpu_sc as plsc

N = 10000
D = 128
E = 320000

LANES = 128
NW = 32
NB = 4
NG = 80
EPAD = NW * NG * LANES
NSUB = 16
NPAD = 10240
ROWS_PER_SUB = NPAD // NSUB

ROWS_BLK = 1000
GRID_N = N // ROWS_BLK


def _segsum_body(feat, y_hbm, src_hbm, dst_hbm, w_hbm, out_hbm,
                 src_v, dst_v, w_v, rows_v, msg_v, *rest):
    K = 5
    gsems = rest[0:K]
    ssems = rest[K:2 * K]
    acc_sh = rest[2 * K]
    y_sh = rest[2 * K + 1]
    c = lax.axis_index("c")
    s = lax.axis_index("s")
    wid = s * 2 + c

    pltpu.sync_copy(src_hbm.at[wid], src_v)
    pltpu.sync_copy(dst_hbm.at[wid], dst_v)
    pltpu.sync_copy(w_hbm.at[wid], w_v)

    zv = jnp.zeros((16,), jnp.float32)
    for r in range(LANES):
        for f0 in range(0, feat, 16):
            msg_v[0, r, pl.ds(f0, 16)] = zv
    for t in range(ROWS_PER_SUB // LANES):
        pltpu.sync_copy(
            msg_v.at[0],
            acc_sh.at[pl.ds(s * ROWS_PER_SUB + t * LANES, LANES)])
    pltpu.sync_copy(y_hbm.at[pl.ds(s * (N // NSUB), N // NSUB)],
                    y_sh.at[pl.ds(s * (N // NSUB), N // NSUB)])
    plsc.subcore_barrier()

    def start_gather(j, b):
        pltpu.async_copy(y_sh.at[src_v.at[j]], rows_v.at[b], gsems[b])

    def wait_gather(j, b):
        pltpu.make_async_copy(y_sh.at[src_v.at[j]], rows_v.at[b], gsems[b]).wait()

    def start_scatter(j, b):
        pltpu.async_copy(msg_v.at[b], acc_sh.at[dst_v.at[j]], ssems[b], add=True)

    def wait_scatter(j, b):
        pltpu.make_async_copy(msg_v.at[b], acc_sh.at[dst_v.at[j]], ssems[b]).wait()

    def scale_rows(j, b):
        def scale(blk, carry2):
            e0 = blk * 16
            wv = w_v[j, pl.ds(e0, 16)]
            for k in range(16):
                we = wv[k]
                for f0 in range(0, feat, 16):
                    msg_v[b, e0 + k, pl.ds(f0, 16)] = (
                        rows_v[b, e0 + k, pl.ds(f0, 16)] * we)
            return carry2
        lax.fori_loop(0, LANES // 16, scale, 0, unroll=4)

    def body(q, carry):
        j0 = q * K
        for b in range(K):
            start_gather(j0 + b, b)
        for b in range(K):
            wait_gather(j0 + b, b)
            scale_rows(j0 + b, b)
        for b in range(K):
            start_scatter(j0 + b, b)
        for b in range(K):
            wait_scatter(j0 + b, b)
        return carry

    lax.fori_loop(0, NG // K, body, 0)
    plsc.subcore_barrier()

    pltpu.sync_copy(acc_sh.at[pl.ds(s * ROWS_PER_SUB, ROWS_PER_SUB)],
                    out_hbm.at[c, pl.ds(s * ROWS_PER_SUB, ROWS_PER_SUB)])


def _make_segsum(feat):
    mesh = plsc.VectorSubcoreMesh(core_axis_name="c", subcore_axis_name="s")
    return pl.kernel(
        functools.partial(_segsum_body, feat),
        out_type=jax.ShapeDtypeStruct((2, NPAD, feat), jnp.float32),
        mesh=mesh,
        compiler_params=pltpu.CompilerParams(use_tc_tiling_on_sc=False),
        scratch_types=[
            pltpu.VMEM((NG + 1, LANES), jnp.int32),
            pltpu.VMEM((NG, LANES), jnp.int32),
            pltpu.VMEM((NG, LANES), jnp.float32),
            pltpu.VMEM((5, LANES, feat), jnp.float32),
            pltpu.VMEM((5, LANES, feat), jnp.float32),
        ] + [pltpu.SemaphoreType.DMA] * 10 + [
            pltpu.VMEM_SHARED((NPAD, feat), jnp.float32),
            pltpu.VMEM_SHARED((N, feat), jnp.float32),
        ],
    )


_segsum32 = _make_segsum(32)
_segsum16 = _make_segsum(16)


def _mm_body(x_ref, w_ref, o_ref):
    o_ref[...] = jnp.dot(x_ref[...], w_ref[...],
                         preferred_element_type=jnp.float32)


def _layer1_matmul(x, w1c):
    return pl.pallas_call(
        _mm_body,
        grid=(GRID_N,),
        in_specs=[pl.BlockSpec((ROWS_BLK, D), lambda i: (i, 0)),
                  pl.BlockSpec((D, 64), lambda i: (0, 0))],
        out_specs=pl.BlockSpec((ROWS_BLK, 64), lambda i: (i, 0)),
        out_shape=jax.ShapeDtypeStruct((N, 64), jnp.float32),
    )(x, w1c)


def _mid_body(a0_ref, a1_ref, r1_ref, b1_ref, w_ref, o_ref):
    h = jnp.maximum(a0_ref[...] + a1_ref[...] + r1_ref[...] + b1_ref[...], 0.0)
    o_ref[...] = jnp.dot(h, w_ref[...], preferred_element_type=jnp.float32)


def _mid_layer(a0, a1, r1, b1, w2c):
    return pl.pallas_call(
        _mid_body,
        grid=(GRID_N,),
        in_specs=[pl.BlockSpec((ROWS_BLK, 32), lambda i: (i, 0)),
                  pl.BlockSpec((ROWS_BLK, 32), lambda i: (i, 0)),
                  pl.BlockSpec((ROWS_BLK, 32), lambda i: (i, 0)),
                  pl.BlockSpec((1, 32), lambda i: (0, 0)),
                  pl.BlockSpec((32, 32), lambda i: (0, 0))],
        out_specs=pl.BlockSpec((ROWS_BLK, 32), lambda i: (i, 0)),
        out_shape=jax.ShapeDtypeStruct((N, 32), jnp.float32),
    )(a0, a1, r1, b1, w2c)


def _final_body(a0_ref, a1_ref, oc_ref, x1_ref, b2_ref, wl_ref, bl_ref,
                out_ref, emb_ref):
    t = (a0_ref[...][:, :8] + a1_ref[...][:, :8]
         + oc_ref[...][:, 16:24] + b2_ref[...])
    m = jnp.max(t, axis=1, keepdims=True)
    lse = jnp.log(jnp.sum(jnp.exp(t - m), axis=1, keepdims=True)) + m
    h2 = t - lse
    emb_ref[...] = h2
    s = jnp.sum(h2 * wl_ref[...][:, :8], axis=1, keepdims=True)
    out = s + x1_ref[...] * wl_ref[...][:, 8:9] + bl_ref[...]
    out_ref[...] = jnp.maximum(out, 0.0)


def _final_layer(a0, a1, oc, x1, b2, wl, bl):
    return pl.pallas_call(
        _final_body,
        grid=(GRID_N,),
        in_specs=[pl.BlockSpec((ROWS_BLK, 16), lambda i: (i, 0)),
                  pl.BlockSpec((ROWS_BLK, 16), lambda i: (i, 0)),
                  pl.BlockSpec((ROWS_BLK, 32), lambda i: (i, 0)),
                  pl.BlockSpec((ROWS_BLK, 1), lambda i: (i, 0)),
                  pl.BlockSpec((1, 8), lambda i: (0, 0)),
                  pl.BlockSpec((1, 9), lambda i: (0, 0)),
                  pl.BlockSpec((1, 1), lambda i: (0, 0))],
        out_specs=[pl.BlockSpec((ROWS_BLK, 1), lambda i: (i, 0)),
                   pl.BlockSpec((ROWS_BLK, 8), lambda i: (i, 0))],
        out_shape=[jax.ShapeDtypeStruct((N, 1), jnp.float32),
                   jax.ShapeDtypeStruct((N, 8), jnp.float32)],
    )(a0, a1, oc, x1, b2, wl, bl)


def kernel(x, edge_index, x1, edge_weight, W1_rel, b1_rel, W1_root,
           W2_rel, b2_rel, W2_root, W_lin, b_lin):
    pad = EPAD - E
    srcp = jnp.concatenate(
        [edge_index[0], jnp.zeros((pad,), jnp.int32)]).reshape(NW, NG, LANES)
    srcp = jnp.concatenate(
        [srcp, jnp.zeros((NW, 1, LANES), jnp.int32)], axis=1)
    dstp = jnp.concatenate(
        [edge_index[1], jnp.zeros((pad,), jnp.int32)]).reshape(NW, NG, LANES)
    wp = jnp.concatenate(
        [edge_weight, jnp.zeros((pad,), jnp.float32)]).reshape(NW, NG, LANES)

    w1c = jnp.concatenate([W1_rel, W1_root], axis=1)
    z1 = _layer1_matmul(x, w1c)
    y1 = z1[:, :32]
    r1 = z1[:, 32:]

    agg1 = _segsum32(y1, srcp, dstp, wp)
    agg1 = agg1[:, :N]

    w2c = jnp.concatenate(
        [W2_rel, jnp.zeros((32, 8), jnp.float32),
         W2_root, jnp.zeros((32, 8), jnp.float32)], axis=1)
    oc = _mid_layer(agg1[0], agg1[1], r1, b1_rel.reshape(1, 32), w2c)
    y2p = oc[:, :16]

    agg2 = _segsum16(y2p, srcp, dstp, wp)
    agg2 = agg2[:, :N]

    out, emb = _final_layer(agg2[0], agg2[1], oc, x1,
                            b2_rel.reshape(1, 8), W_lin.T,
                            b_lin.reshape(1, 1))
    return (out, emb)


# speedup vs baseline: 1.6334x; 1.0059x over previous
import functools

import jax
import jax.numpy as jnp
from jax import lax
from jax.experimental import pallas as pl
from jax.experimental.pallas import tpu as pltpu
from jax.experimental.pallas import tpu_sc as plsc

N = 10000
D = 128
E = 320000

LANES = 128
NW = 32
NB = 4
NG = 80
EPAD = NW * NG * LANES
NSUB = 16
NPAD = 10240
ROWS_PER_SUB = NPAD // NSUB

ROWS_BLK = 1000
GRID_N = N // ROWS_BLK


def _segsum_body(feat, y_hbm, src_hbm, dst_hbm, w_hbm, out_hbm,
                 src_v, dst_v, w_v, rows_v, msg_v, *rest):
    K = 5 if feat == 32 else 10
    gsems = rest[0:K]
    ssems = rest[K:2 * K]
    acc_sh = rest[2 * K]
    y_sh = rest[2 * K + 1]
    c = lax.axis_index("c")
    s = lax.axis_index("s")
    wid = s * 2 + c

    pltpu.sync_copy(src_hbm.at[wid], src_v)
    pltpu.sync_copy(dst_hbm.at[wid], dst_v)
    pltpu.sync_copy(w_hbm.at[wid], w_v)

    zv = jnp.zeros((16,), jnp.float32)
    for r in range(LANES):
        for f0 in range(0, feat, 16):
            msg_v[0, r, pl.ds(f0, 16)] = zv
    for t in range(ROWS_PER_SUB // LANES):
        pltpu.sync_copy(
            msg_v.at[0],
            acc_sh.at[pl.ds(s * ROWS_PER_SUB + t * LANES, LANES)])
    pltpu.sync_copy(y_hbm.at[pl.ds(s * (N // NSUB), N // NSUB)],
                    y_sh.at[pl.ds(s * (N // NSUB), N // NSUB)])
    plsc.subcore_barrier()

    def start_gather(j, b):
        pltpu.async_copy(y_sh.at[src_v.at[j]], rows_v.at[b], gsems[b])

    def wait_gather(j, b):
        pltpu.make_async_copy(y_sh.at[src_v.at[j]], rows_v.at[b], gsems[b]).wait()

    def start_scatter(j, b):
        pltpu.async_copy(msg_v.at[b], acc_sh.at[dst_v.at[j]], ssems[b], add=True)

    def wait_scatter(j, b):
        pltpu.make_async_copy(msg_v.at[b], acc_sh.at[dst_v.at[j]], ssems[b]).wait()

    def scale_rows(j, b):
        def scale(blk, carry2):
            e0 = blk * 16
            wv = w_v[j, pl.ds(e0, 16)]
            for k in range(16):
                we = wv[k]
                for f0 in range(0, feat, 16):
                    msg_v[b, e0 + k, pl.ds(f0, 16)] = (
                        rows_v[b, e0 + k, pl.ds(f0, 16)] * we)
            return carry2
        lax.fori_loop(0, LANES // 16, scale, 0, unroll=4)

    def body(q, carry):
        j0 = q * K
        for b in range(K):
            start_gather(j0 + b, b)
        for b in range(K):
            wait_gather(j0 + b, b)
            scale_rows(j0 + b, b)
        for b in range(K):
            start_scatter(j0 + b, b)
        for b in range(K):
            wait_scatter(j0 + b, b)
        return carry

    lax.fori_loop(0, NG // K, body, 0)
    plsc.subcore_barrier()

    pltpu.sync_copy(acc_sh.at[pl.ds(s * ROWS_PER_SUB, ROWS_PER_SUB)],
                    out_hbm.at[c, pl.ds(s * ROWS_PER_SUB, ROWS_PER_SUB)])


def _make_segsum(feat):
    mesh = plsc.VectorSubcoreMesh(core_axis_name="c", subcore_axis_name="s")
    return pl.kernel(
        functools.partial(_segsum_body, feat),
        out_type=jax.ShapeDtypeStruct((2, NPAD, feat), jnp.float32),
        mesh=mesh,
        compiler_params=pltpu.CompilerParams(use_tc_tiling_on_sc=False),
        scratch_types=[
            pltpu.VMEM((NG + 1, LANES), jnp.int32),
            pltpu.VMEM((NG, LANES), jnp.int32),
            pltpu.VMEM((NG, LANES), jnp.float32),
            pltpu.VMEM((5 if feat == 32 else 10, LANES, feat), jnp.float32),
            pltpu.VMEM((5 if feat == 32 else 10, LANES, feat), jnp.float32),
        ] + [pltpu.SemaphoreType.DMA] * (10 if feat == 32 else 20) + [
            pltpu.VMEM_SHARED((NPAD, feat), jnp.float32),
            pltpu.VMEM_SHARED((N, feat), jnp.float32),
        ],
    )


_segsum32 = _make_segsum(32)
_segsum16 = _make_segsum(16)


def _mm_body(x_ref, w_ref, o_ref):
    o_ref[...] = jnp.dot(x_ref[...], w_ref[...],
                         preferred_element_type=jnp.float32)


def _layer1_matmul(x, w1c):
    return pl.pallas_call(
        _mm_body,
        grid=(GRID_N,),
        in_specs=[pl.BlockSpec((ROWS_BLK, D), lambda i: (i, 0)),
                  pl.BlockSpec((D, 64), lambda i: (0, 0))],
        out_specs=pl.BlockSpec((ROWS_BLK, 64), lambda i: (i, 0)),
        out_shape=jax.ShapeDtypeStruct((N, 64), jnp.float32),
    )(x, w1c)


def _mid_body(a0_ref, a1_ref, r1_ref, b1_ref, w_ref, o_ref):
    h = jnp.maximum(a0_ref[...] + a1_ref[...] + r1_ref[...] + b1_ref[...], 0.0)
    o_ref[...] = jnp.dot(h, w_ref[...], preferred_element_type=jnp.float32)


def _mid_layer(a0, a1, r1, b1, w2c):
    return pl.pallas_call(
        _mid_body,
        grid=(GRID_N,),
        in_specs=[pl.BlockSpec((ROWS_BLK, 32), lambda i: (i, 0)),
                  pl.BlockSpec((ROWS_BLK, 32), lambda i: (i, 0)),
                  pl.BlockSpec((ROWS_BLK, 32), lambda i: (i, 0)),
                  pl.BlockSpec((1, 32), lambda i: (0, 0)),
                  pl.BlockSpec((32, 32), lambda i: (0, 0))],
        out_specs=pl.BlockSpec((ROWS_BLK, 32), lambda i: (i, 0)),
        out_shape=jax.ShapeDtypeStruct((N, 32), jnp.float32),
    )(a0, a1, r1, b1, w2c)


def _final_body(a0_ref, a1_ref, oc_ref, x1_ref, b2_ref, wl_ref, bl_ref,
                out_ref, emb_ref):
    t = (a0_ref[...][:, :8] + a1_ref[...][:, :8]
         + oc_ref[...][:, 16:24] + b2_ref[...])
    m = jnp.max(t, axis=1, keepdims=True)
    lse = jnp.log(jnp.sum(jnp.exp(t - m), axis=1, keepdims=True)) + m
    h2 = t - lse
    emb_ref[...] = h2
    s = jnp.sum(h2 * wl_ref[...][:, :8], axis=1, keepdims=True)
    out = s + x1_ref[...] * wl_ref[...][:, 8:9] + bl_ref[...]
    out_ref[...] = jnp.maximum(out, 0.0)


def _final_layer(a0, a1, oc, x1, b2, wl, bl):
    return pl.pallas_call(
        _final_body,
        grid=(GRID_N,),
        in_specs=[pl.BlockSpec((ROWS_BLK, 16), lambda i: (i, 0)),
                  pl.BlockSpec((ROWS_BLK, 16), lambda i: (i, 0)),
                  pl.BlockSpec((ROWS_BLK, 32), lambda i: (i, 0)),
                  pl.BlockSpec((ROWS_BLK, 1), lambda i: (i, 0)),
                  pl.BlockSpec((1, 8), lambda i: (0, 0)),
                  pl.BlockSpec((1, 9), lambda i: (0, 0)),
                  pl.BlockSpec((1, 1), lambda i: (0, 0))],
        out_specs=[pl.BlockSpec((ROWS_BLK, 1), lambda i: (i, 0)),
                   pl.BlockSpec((ROWS_BLK, 8), lambda i: (i, 0))],
        out_shape=[jax.ShapeDtypeStruct((N, 1), jnp.float32),
                   jax.ShapeDtypeStruct((N, 8), jnp.float32)],
    )(a0, a1, oc, x1, b2, wl, bl)


def kernel(x, edge_index, x1, edge_weight, W1_rel, b1_rel, W1_root,
           W2_rel, b2_rel, W2_root, W_lin, b_lin):
    pad = EPAD - E
    srcp = jnp.concatenate(
        [edge_index[0], jnp.zeros((pad,), jnp.int32)]).reshape(NW, NG, LANES)
    srcp = jnp.concatenate(
        [srcp, jnp.zeros((NW, 1, LANES), jnp.int32)], axis=1)
    dstp = jnp.concatenate(
        [edge_index[1], jnp.zeros((pad,), jnp.int32)]).reshape(NW, NG, LANES)
    wp = jnp.concatenate(
        [edge_weight, jnp.zeros((pad,), jnp.float32)]).reshape(NW, NG, LANES)

    w1c = jnp.concatenate([W1_rel, W1_root], axis=1)
    z1 = _layer1_matmul(x, w1c)
    y1 = z1[:, :32]
    r1 = z1[:, 32:]

    agg1 = _segsum32(y1, srcp, dstp, wp)
    agg1 = agg1[:, :N]

    w2c = jnp.concatenate(
        [W2_rel, jnp.zeros((32, 8), jnp.float32),
         W2_root, jnp.zeros((32, 8), jnp.float32)], axis=1)
    oc = _mid_layer(agg1[0], agg1[1], r1, b1_rel.reshape(1, 32), w2c)
    y2p = oc[:, :16]

    agg2 = _segsum16(y2p, srcp, dstp, wp)
    agg2 = agg2[:, :N]

    out, emb = _final_layer(agg2[0], agg2[1], oc, x1,
                            b2_rel.reshape(1, 8), W_lin.T,
                            b_lin.reshape(1, 1))
    return (out, emb)
